# Initial kernel scaffold; baseline (speedup 1.0000x reference)
import jax
import jax.numpy as jnp
from jax import lax
from jax.experimental import pallas as pl
from jax.experimental.pallas import tpu as pltpu
from jax.experimental.pallas import tpu_sc as plsc

NC = 2
NS = 16
L = 16
NW = NC * NS
CHUNK = 128
HID = 128
SEQ = 200
PAD_ID = 0


def _build(n_tok):
    per_w = n_tok // NW
    n_chunks = per_w // CHUNK
    mesh = plsc.VectorSubcoreMesh(core_axis_name="c", subcore_axis_name="s")

    def body(enc_ids, dec_ids, src_tab, trg_tab, pos_tab, enc_out, dec_out,
             idx_v, prow_v, tok_v, pos_v, sem_t, sem_p):
        wid = lax.axis_index("s") * NC + lax.axis_index("c")

        for ids_hbm, tab_hbm, out_hbm in ((enc_ids, src_tab, enc_out),
                                          (dec_ids, trg_tab, dec_out)):
            @pl.loop(0, n_chunks)
            def _chunk(c, ids_hbm=ids_hbm, tab_hbm=tab_hbm, out_hbm=out_hbm):
                base = wid * per_w + c * CHUNK
                pltpu.sync_copy(ids_hbm.at[pl.ds(base, CHUNK)], idx_v)
                cp_t = pltpu.async_copy(tab_hbm.at[idx_v], tok_v, sem_t)
                for g in range(CHUNK // L):
                    ids16 = idx_v[pl.ds(g * L, L)]
                    f16 = base + g * L + lax.iota(jnp.int32, (L,))
                    t16 = lax.rem(f16, SEQ)
                    prow_v[pl.ds(g * L, L)] = jnp.where(ids16 == PAD_ID, 0, t16 + 1)
                cp_p = pltpu.async_copy(pos_tab.at[prow_v], pos_v, sem_p)
                cp_t.wait()
                cp_p.wait()

                @pl.loop(0, CHUNK)
                def _tok(i):
                    for j in range(HID // L):
                        sl = pl.ds(j * L, L)
                        plsc.addupdate(tok_v.at[i, sl], pos_v[i, sl])

                pltpu.sync_copy(tok_v, out_hbm.at[pl.ds(base, CHUNK)])

    return pl.kernel(
        body,
        out_type=(jax.ShapeDtypeStruct((n_tok, HID), jnp.float32),
                  jax.ShapeDtypeStruct((n_tok, HID), jnp.float32)),
        mesh=mesh,
        scratch_types=[
            pltpu.VMEM((CHUNK,), jnp.int32),
            pltpu.VMEM((CHUNK,), jnp.int32),
            pltpu.VMEM((CHUNK, HID), jnp.float32),
            pltpu.VMEM((CHUNK, HID), jnp.float32),
            pltpu.SemaphoreType.DMA,
            pltpu.SemaphoreType.DMA,
        ],
    )


def kernel(enc_ids, dec_ids, src_table, trg_table, pos_table):
    B, T = enc_ids.shape
    n_tok = B * T
    enc_flat = enc_ids.astype(jnp.int32).reshape(n_tok)
    dec_flat = dec_ids.astype(jnp.int32).reshape(n_tok)
    enc_o, dec_o = _build(n_tok)(enc_flat, dec_flat, src_table, trg_table,
                                 pos_table)
    return enc_o.reshape(B, T, HID), dec_o.reshape(B, T, HID)


# speedup vs baseline: 5.5699x; 5.5699x over previous
import jax
import jax.numpy as jnp
from jax import lax
from jax.experimental import pallas as pl
from jax.experimental.pallas import tpu as pltpu
from jax.experimental.pallas import tpu_sc as plsc

NC = 2
NS = 16
L = 16
NW = NC * NS
CHUNK = 128
HID = 128
SEQ = 200
PAD_ID = 0


def _build(n_tok):
    per_w = n_tok // NW
    n_chunks = per_w // CHUNK
    mesh = plsc.VectorSubcoreMesh(core_axis_name="c", subcore_axis_name="s")

    def body(enc_ids, dec_ids, src_tab, trg_tab, pos_tab, enc_out, dec_out,
             idx_v, prow_v, tok_v, pos_v, sem_t, sem_p):
        wid = lax.axis_index("s") * NC + lax.axis_index("c")

        for ids_hbm, tab_hbm, out_hbm in ((enc_ids, src_tab, enc_out),
                                          (dec_ids, trg_tab, dec_out)):
            @pl.loop(0, n_chunks)
            def _chunk(c, ids_hbm=ids_hbm, tab_hbm=tab_hbm, out_hbm=out_hbm):
                base = wid * per_w + c * CHUNK
                pltpu.sync_copy(ids_hbm.at[pl.ds(base, CHUNK)], idx_v)
                cp_t = pltpu.async_copy(tab_hbm.at[idx_v], tok_v, sem_t)
                for g in range(CHUNK // L):
                    ids16 = idx_v[pl.ds(g * L, L)]
                    f16 = base + g * L + lax.iota(jnp.int32, L)
                    t16 = lax.rem(f16, SEQ)
                    prow_v[pl.ds(g * L, L)] = jnp.where(ids16 == PAD_ID, 0, t16 + 1)
                cp_p = pltpu.async_copy(pos_tab.at[prow_v], pos_v, sem_p)
                cp_t.wait()
                cp_p.wait()

                @pl.loop(0, CHUNK)
                def _tok(i):
                    for j in range(HID // L):
                        sl = pl.ds(j * L, L)
                        plsc.addupdate(tok_v.at[i, sl], pos_v[i, sl])

                pltpu.sync_copy(tok_v, out_hbm.at[pl.ds(base, CHUNK)])

    return pl.kernel(
        body,
        out_type=(jax.ShapeDtypeStruct((n_tok, HID), jnp.float32),
                  jax.ShapeDtypeStruct((n_tok, HID), jnp.float32)),
        mesh=mesh,
        scratch_types=[
            pltpu.VMEM((CHUNK,), jnp.int32),
            pltpu.VMEM((CHUNK,), jnp.int32),
            pltpu.VMEM((CHUNK, HID), jnp.float32),
            pltpu.VMEM((CHUNK, HID), jnp.float32),
            pltpu.SemaphoreType.DMA,
            pltpu.SemaphoreType.DMA,
        ],
    )


def kernel(enc_ids, dec_ids, src_table, trg_table, pos_table):
    B, T = enc_ids.shape
    n_tok = B * T
    enc_flat = enc_ids.astype(jnp.int32).reshape(n_tok)
    dec_flat = dec_ids.astype(jnp.int32).reshape(n_tok)
    enc_o, dec_o = _build(n_tok)(enc_flat, dec_flat, src_table, trg_table,
                                 pos_table)
    return enc_o.reshape(B, T, HID), dec_o.reshape(B, T, HID)


# speedup vs baseline: 5.8049x; 1.0422x over previous
import jax
import jax.numpy as jnp
from jax import lax
from jax.experimental import pallas as pl
from jax.experimental.pallas import tpu as pltpu
from jax.experimental.pallas import tpu_sc as plsc

NC = 2
NS = 16
L = 16
NW = NC * NS
CHUNK = 128
HID = 128
SEQ = 200
PAD_ID = 0


def _build(n_tok):
    per_w = n_tok // NW
    cpw = per_w // CHUNK
    assert cpw % 2 == 0
    mesh = plsc.VectorSubcoreMesh(core_axis_name="c", subcore_axis_name="s")

    def body(enc_ids, dec_ids, src_tab, trg_tab, pos_tab, enc_out, dec_out,
             idx_big, prow0, prow1, tok0, tok1, pos0, pos1, out0, out1,
             sem_t0, sem_t1, sem_p0, sem_p1, sem_o0, sem_o1):
        wid = lax.axis_index("s") * NC + lax.axis_index("c")
        prow = (prow0, prow1)
        tok = (tok0, tok1)
        pos = (pos0, pos1)
        out = (out0, out1)
        sem_t = (sem_t0, sem_t1)
        sem_p = (sem_p0, sem_p1)
        sem_o = (sem_o0, sem_o1)

        for ids_hbm, tab_hbm, out_hbm in ((enc_ids, src_tab, enc_out),
                                          (dec_ids, trg_tab, dec_out)):
            pltpu.sync_copy(ids_hbm.at[pl.ds(wid * per_w, per_w)], idx_big)

            def issue(c, s):
                pltpu.async_copy(tab_hbm.at[idx_big.at[pl.ds(c * CHUNK, CHUNK)]],
                                 tok[s], sem_t[s])
                base = (wid * cpw + c) * CHUNK
                for g in range(CHUNK // L):
                    ids16 = idx_big[pl.ds(c * CHUNK + g * L, L)]
                    f16 = base + g * L + lax.iota(jnp.int32, L)
                    t16 = lax.rem(f16, SEQ)
                    prow[s][pl.ds(g * L, L)] = jnp.where(ids16 == PAD_ID, 0,
                                                         t16 + 1)
                pltpu.async_copy(pos_tab.at[prow[s]], pos[s], sem_p[s])

            def consume(c, s):
                pltpu.make_async_copy(tab_hbm.at[idx_big.at[pl.ds(c * CHUNK,
                                                                  CHUNK)]],
                                      tok[s], sem_t[s]).wait()
                pltpu.make_async_copy(pos_tab.at[prow[s]], pos[s],
                                      sem_p[s]).wait()
                base = (wid * cpw + c) * CHUNK

                @pl.when(c > 1)
                def _():
                    pltpu.make_async_copy(out[s], out_hbm.at[pl.ds(base, CHUNK)],
                                          sem_o[s]).wait()

                @pl.loop(0, CHUNK)
                def _tok(i):
                    for j in range(HID // L):
                        sl = pl.ds(j * L, L)
                        out[s][i, sl] = tok[s][i, sl] + pos[s][i, sl]

                pltpu.async_copy(out[s], out_hbm.at[pl.ds(base, CHUNK)],
                                 sem_o[s])

            issue(0, 0)
            issue(1, 1)

            @pl.loop(0, cpw, step=2)
            def _chunks(c):
                consume(c, 0)

                @pl.when(c + 2 < cpw)
                def _():
                    issue(c + 2, 0)

                consume(c + 1, 1)

                @pl.when(c + 3 < cpw)
                def _():
                    issue(c + 3, 1)

            for s in (0, 1):
                pltpu.make_async_copy(out[s], out_hbm.at[pl.ds(0, CHUNK)],
                                      sem_o[s]).wait()

    return pl.kernel(
        body,
        out_type=(jax.ShapeDtypeStruct((n_tok, HID), jnp.float32),
                  jax.ShapeDtypeStruct((n_tok, HID), jnp.float32)),
        mesh=mesh,
        scratch_types=[
            pltpu.VMEM((n_tok // NW,), jnp.int32),
            pltpu.VMEM((CHUNK,), jnp.int32),
            pltpu.VMEM((CHUNK,), jnp.int32),
            pltpu.VMEM((CHUNK, HID), jnp.float32),
            pltpu.VMEM((CHUNK, HID), jnp.float32),
            pltpu.VMEM((CHUNK, HID), jnp.float32),
            pltpu.VMEM((CHUNK, HID), jnp.float32),
            pltpu.VMEM((CHUNK, HID), jnp.float32),
            pltpu.VMEM((CHUNK, HID), jnp.float32),
            pltpu.SemaphoreType.DMA,
            pltpu.SemaphoreType.DMA,
            pltpu.SemaphoreType.DMA,
            pltpu.SemaphoreType.DMA,
            pltpu.SemaphoreType.DMA,
            pltpu.SemaphoreType.DMA,
        ],
    )


def kernel(enc_ids, dec_ids, src_table, trg_table, pos_table):
    B, T = enc_ids.shape
    n_tok = B * T
    enc_flat = enc_ids.astype(jnp.int32).reshape(n_tok)
    dec_flat = dec_ids.astype(jnp.int32).reshape(n_tok)
    enc_o, dec_o = _build(n_tok)(enc_flat, dec_flat, src_table, trg_table,
                                 pos_table)
    return enc_o.reshape(B, T, HID), dec_o.reshape(B, T, HID)
